# baseline (speedup 1.0000x reference)
import functools

import jax
import jax.numpy as jnp
from jax import lax
from jax.experimental import pallas as pl
from jax.experimental.pallas import tpu as pltpu
from jax.experimental.pallas import tpu_sc as plsc

N = 10000
E = 160000
D_IN = 500
HID = 1000
HEADS = 8
D_OUT = 200
B = 1024

ET = E + N
EC = 64
ETP = ((ET + EC - 1) // EC) * EC
DW1 = 512
DW2 = 256


def _make_sc_agg(H, DW, R, NC, NS):
    NW = NC * NS
    ACC = H * DW
    mesh = plsc.VectorSubcoreMesh(core_axis_name="c", subcore_axis_name="s")

    @functools.partial(
        pl.kernel,
        out_type=jax.ShapeDtypeStruct((R, ACC), jnp.float32),
        mesh=mesh,
        scratch_types=[
            pltpu.VMEM((EC,), jnp.int32),
            pltpu.VMEM((EC + 16,), jnp.int32),
            pltpu.VMEM((H, EC + 16), jnp.float32),
            pltpu.VMEM((EC, DW), jnp.float32),
            pltpu.VMEM((ACC,), jnp.float32),
            pltpu.VMEM((48,), jnp.int32),
            pltpu.SemaphoreType.DMA,
        ],
    )
    def k(src_hbm, dst_hbm, ex_hbm, feat_hbm, woff_hbm, out_hbm,
          srcv, dstv, exv, rowsv, accv, woffv, sem):
        wid = lax.axis_index("s") * NC + lax.axis_index("c")
        pltpu.sync_copy(woff_hbm, woffv)
        w0 = woffv[pl.ds(wid, 16)][0]
        w1 = woffv[pl.ds(wid + 1, 16)][0]

        zero16 = jnp.zeros((16,), jnp.float32)
        for j in range(ACC // 16):
            accv[pl.ds(j * 16, 16)] = zero16

        def flush(d):
            pltpu.sync_copy(accv, out_hbm.at[d])
            for j in range(ACC // 16):
                accv[pl.ds(j * 16, 16)] = zero16

        c0 = (w0 // EC) * EC
        nch = (w1 - c0 + EC - 1) // EC

        def chunk_body(ci, cur_d):
            c = c0 + ci * EC
            pltpu.sync_copy(src_hbm.at[pl.ds(c, EC)], srcv)
            pltpu.sync_copy(dst_hbm.at[pl.ds(c, EC)], dstv.at[pl.ds(0, EC)])
            pltpu.sync_copy(ex_hbm.at[:, pl.ds(c, EC)], exv.at[:, pl.ds(0, EC)])
            pltpu.async_copy(feat_hbm.at[srcv], rowsv, sem).wait()

            def edge_body(i, cur_d):
                g = c + i
                d = dstv[pl.ds(i, 16)][0]

                def process(cd):
                    def do_flush(cd):
                        flush(cd)
                        return cd
                    lax.cond((d != cd) & (cd >= 0), do_flush, lambda x: x, cd)
                    for h in range(H):
                        s = exv[h, pl.ds(i, 16)][0]
                        for j in range(DW // 16):
                            sl = pl.ds(h * DW + j * 16, 16)
                            accv[sl] = accv[sl] + s * rowsv[i, pl.ds(j * 16, 16)]
                    return d

                return lax.cond((g >= w0) & (g < w1), process,
                                lambda cd: cd, cur_d)

            return lax.fori_loop(0, EC, edge_body, cur_d)

        cur_d = lax.fori_loop(0, nch, chunk_body, jnp.int32(-1))
        lax.cond(cur_d >= 0, lambda d: (flush(d), d)[1], lambda d: d, cur_d)

    return k


def _logits_kernel(x_ref, v_ref, o_ref):
    o_ref[...] = jnp.dot(x_ref[...], v_ref[...],
                         preferred_element_type=jnp.float32)


def _dense_kernel(agg_ref, w1_ref, b1_ref, w2_ref, o_ref):
    blk = agg_ref[...]
    outs = []
    for h in range(HEADS):
        a = blk[:, h * DW1:h * DW1 + D_IN]
        s = blk[:, h * DW1 + D_IN:h * DW1 + D_IN + 1]
        a = a / (s + 1e-16)
        outs.append(jnp.dot(a, w1_ref[:, h * HID:(h + 1) * HID],
                            preferred_element_type=jnp.float32))
    h1 = jnp.concatenate(outs, axis=1) + b1_ref[...]
    h1 = jnp.where(h1 > 0, h1, jnp.exp(h1) - 1.0)
    xt2 = jnp.dot(h1, w2_ref[...], preferred_element_type=jnp.float32)
    col = lax.broadcasted_iota(jnp.int32, xt2.shape, 1)
    o_ref[...] = jnp.where(col == D_OUT, 1.0, xt2)


def _mlp_kernel(r1_ref, r2_ref, wl1_ref, bl1_ref, wl2_ref, bl2_ref,
                wf1_ref, bf1_ref, sc1_ref, sh1_ref, wf2_ref, bf2_ref,
                sc2_ref, sh2_ref, wf3_ref, bf3_ref, o_ref):
    def norm(r):
        return r[:, :D_OUT] / (r[:, D_OUT:D_OUT + 1] + 1e-16)

    z1 = jnp.dot(norm(r1_ref[...]), wl1_ref[...],
                 preferred_element_type=jnp.float32) + bl1_ref[...]
    z2 = jnp.dot(norm(r2_ref[...]), wl2_ref[...],
                 preferred_element_type=jnp.float32) + bl2_ref[...]
    z = jnp.concatenate([z1, z2], axis=1)
    z = jnp.maximum(jnp.dot(z, wf1_ref[...],
                            preferred_element_type=jnp.float32)
                    + bf1_ref[...], 0.0)
    z = z * sc1_ref[...] + sh1_ref[...]
    z = jnp.maximum(jnp.dot(z, wf2_ref[...],
                            preferred_element_type=jnp.float32)
                    + bf2_ref[...], 0.0)
    z = z * sc2_ref[...] + sh2_ref[...]
    z = jnp.dot(z, wf3_ref[...], preferred_element_type=jnp.float32) \
        + bf3_ref[...]
    m = jnp.max(z, axis=1, keepdims=True)
    ez = jnp.exp(z - m)
    o_ref[...] = ez / jnp.sum(ez, axis=1, keepdims=True)


def _leaky_exp(v):
    return jnp.exp(jnp.where(v > 0, v, 0.2 * v))


def kernel(x1_id, x2_id, edge_index, x, W1, att_src1, att_dst1, b1, W2,
           att_src2, att_dst2, b2, Wl1, bl1, Wl2, bl2, Wf1, bf1, g1, be1,
           rm1, rv1, Wf2, bf2, g2, be2, rm2, rv2, Wf3, bf3):
    f32 = jnp.float32
    info = plsc.get_sparse_core_info()
    NC, NS = info.num_cores, info.num_subcores
    NW = NC * NS
    NPW = -(-N // NW)
    R = NW * NPW

    loops = jnp.arange(N, dtype=jnp.int32)
    src = jnp.concatenate([edge_index[0].astype(jnp.int32), loops])
    dst = jnp.concatenate([edge_index[1].astype(jnp.int32), loops])
    order = jnp.argsort(dst)
    src_s = src[order]
    dst_s = dst[order]
    pad = ETP - ET
    src_p = jnp.concatenate([src_s, jnp.zeros((pad,), jnp.int32)])
    dst_p = jnp.concatenate([dst_s, jnp.zeros((pad,), jnp.int32)])
    woff = jnp.searchsorted(dst_s, NPW * jnp.arange(NW + 1)).astype(jnp.int32)
    woff = jnp.concatenate([woff, jnp.zeros((48 - NW - 1,), jnp.int32)])

    W1h = W1.reshape(D_IN, HEADS, HID)
    V1 = jnp.concatenate([
        jnp.einsum("dhc,hc->dh", W1h, att_src1),
        jnp.einsum("dhc,hc->dh", W1h, att_dst1)], axis=1)
    vs2 = (W2 @ att_src2[0])[:, None]
    vd2 = (W2 @ att_dst2[0])[:, None]
    W2p = jnp.concatenate([W2, jnp.zeros((HEADS * HID, 1), f32), vs2, vd2,
                           jnp.zeros((HEADS * HID, DW2 - D_OUT - 3), f32)],
                          axis=1)
    bl1p = b2 @ Wl1 + bl1
    bl2p = b2 @ Wl2 + bl2
    sc1 = g1 / jnp.sqrt(rv1 + 1e-5)
    sh1 = be1 - rm1 * sc1
    sc2 = g2 / jnp.sqrt(rv2 + 1e-5)
    sh2 = be2 - rm2 * sc2

    a1 = pl.pallas_call(
        _logits_kernel,
        grid=(5,),
        in_specs=[pl.BlockSpec((2000, D_IN), lambda i: (i, 0)),
                  pl.BlockSpec((D_IN, 2 * HEADS), lambda i: (0, 0))],
        out_specs=pl.BlockSpec((2000, 2 * HEADS), lambda i: (i, 0)),
        out_shape=jax.ShapeDtypeStruct((N, 2 * HEADS), f32),
    )(x, V1)
    ex1 = _leaky_exp(a1[src_s, :HEADS] + a1[dst_s, HEADS:])
    ex1 = jnp.concatenate([ex1, jnp.zeros((pad, HEADS), f32)]).T
    ex1 = jnp.asarray(ex1, f32)

    x_aug = jnp.concatenate(
        [x, jnp.ones((N, 1), f32), jnp.zeros((N, DW1 - D_IN - 1), f32)],
        axis=1)

    agg1 = _make_sc_agg(HEADS, DW1, R, NC, NS)(
        src_p, dst_p, ex1, x_aug, woff)[:N]

    xt2 = pl.pallas_call(
        _dense_kernel,
        grid=(50,),
        in_specs=[pl.BlockSpec((200, HEADS * DW1), lambda i: (i, 0)),
                  pl.BlockSpec((D_IN, HEADS * HID), lambda i: (0, 0)),
                  pl.BlockSpec((1, HEADS * HID), lambda i: (0, 0)),
                  pl.BlockSpec((HEADS * HID, DW2), lambda i: (0, 0))],
        out_specs=pl.BlockSpec((200, DW2), lambda i: (i, 0)),
        out_shape=jax.ShapeDtypeStruct((N, DW2), f32),
    )(agg1, W1, b1.reshape(1, -1), W2p)

    ex2 = _leaky_exp(xt2[src_s, D_OUT + 1] + xt2[dst_s, D_OUT + 2])
    ex2 = jnp.concatenate([ex2, jnp.zeros((pad,), f32)])[None, :]
    ex2 = jnp.asarray(ex2, f32)
    out2 = _make_sc_agg(1, DW2, R, NC, NS)(
        src_p, dst_p, ex2, xt2, woff)

    r1 = out2[x1_id]
    r2 = out2[x2_id]
    full = lambda s: pl.BlockSpec(s, lambda: tuple(0 for _ in s))
    z = pl.pallas_call(
        _mlp_kernel,
        in_specs=[full((B, DW2)), full((B, DW2)),
                  full((D_OUT, 250)), full((1, 250)),
                  full((D_OUT, 250)), full((1, 250)),
                  full((D_IN, 1000)), full((1, 1000)),
                  full((1, 1000)), full((1, 1000)),
                  full((1000, D_IN)), full((1, D_IN)),
                  full((1, D_IN)), full((1, D_IN)),
                  full((D_IN, 2)), full((1, 2))],
        out_specs=full((B, 2)),
        out_shape=jax.ShapeDtypeStruct((B, 2), f32),
    )(r1, r2, Wl1, bl1p.reshape(1, -1), Wl2, bl2p.reshape(1, -1),
      Wf1, bf1.reshape(1, -1), sc1.reshape(1, -1), sh1.reshape(1, -1),
      Wf2, bf2.reshape(1, -1), sc2.reshape(1, -1), sh2.reshape(1, -1),
      Wf3, bf3.reshape(1, -1))
    return z
